# Initial kernel scaffold; baseline (speedup 1.0000x reference)
import jax
import jax.numpy as jnp
from jax.experimental import pallas as pl


def kernel(x, mask, logdet):
    raise NotImplementedError("write your pallas kernel here")


# speedup vs baseline: 2.4911x; 2.4911x over previous
import jax
import jax.numpy as jnp
from jax import lax
from jax.experimental import pallas as pl
from jax.experimental.pallas import tpu as pltpu
from jax.experimental.pallas import tpu_sc as plsc

_B, _T = 16, 4096
_L = 16


def _flip_compact_body(x_hbm, out_hbm, xbuf, obuf):
    c = lax.axis_index("c")
    s = lax.axis_index("s")
    wid = s * 2 + c

    @pl.when(wid < _B)
    def _():
        row = wid
        pltpu.sync_copy(x_hbm.at[row], xbuf)

        zeros = jnp.zeros((_L,), jnp.float32)

        def zero_step(i, carry):
            obuf[pl.ds(i * _L, _L)] = zeros
            return carry

        lax.fori_loop(0, _T // _L, zero_step, jnp.int32(0))

        def compact_step(i, off):
            v = xbuf[pl.ds(_T - (i + 1) * _L, _L)]
            v = lax.rev(v, (0,))
            m = v != 0.0
            mi = m.astype(jnp.int32)
            pos = jnp.full((_L,), off, jnp.int32) + (plsc.cumsum(mi) - 1)
            plsc.store_scatter(obuf, [pos], v, mask=m)
            return off + jnp.sum(mi)

        lax.fori_loop(0, _T // _L, compact_step, jnp.int32(0))

        pltpu.sync_copy(obuf, out_hbm.at[row])


def kernel(x, mask, logdet):
    mesh = plsc.VectorSubcoreMesh(core_axis_name="c", subcore_axis_name="s")
    f = pl.kernel(
        _flip_compact_body,
        out_type=jax.ShapeDtypeStruct((_B, _T), jnp.float32),
        mesh=mesh,
        scratch_types=[
            pltpu.VMEM((_T,), jnp.float32),
            pltpu.VMEM((_T,), jnp.float32),
        ],
        compiler_params=pltpu.CompilerParams(needs_layout_passes=False),
    )
    x_ = f(x)
    return (x_, mask, logdet)


# speedup vs baseline: 2.6184x; 1.0511x over previous
import jax
import jax.numpy as jnp
from jax import lax
from jax.experimental import pallas as pl
from jax.experimental.pallas import tpu as pltpu
from jax.experimental.pallas import tpu_sc as plsc

_B, _T = 16, 4096
_L = 16


def _flip_compact_body(x_hbm, out_hbm, xbuf, obuf):
    c = lax.axis_index("c")
    s = lax.axis_index("s")
    wid = s * 2 + c

    @pl.when(wid < _B)
    def _():
        row = wid
        pltpu.sync_copy(x_hbm.at[row], xbuf)

        iota = lax.iota(jnp.int32, _L)
        lanes = jnp.full((_L,), _L, jnp.int32)
        tvec = jnp.full((_L,), _T, jnp.int32)

        def compact_step(i, carry):
            off, zoff = carry
            v = xbuf[pl.ds(_T - (i + 1) * _L, _L)]
            m = v != 0.0
            mi = m.astype(jnp.int32)
            csum = plsc.cumsum(mi)
            cnt = plsc.all_reduce_population_count(m)
            zcnt = lanes - cnt
            a = off + cnt
            b = tvec - zoff - zcnt + iota
            pos = jnp.where(m, a, b) - csum
            plsc.store_scatter(obuf, [pos], v)
            return (off + cnt, zoff + zcnt)

        zero = jnp.zeros((_L,), jnp.int32)
        lax.fori_loop(0, _T // _L, compact_step, (zero, zero))
        pltpu.sync_copy(obuf, out_hbm.at[row])


def kernel(x, mask, logdet):
    mesh = plsc.VectorSubcoreMesh(core_axis_name="c", subcore_axis_name="s")
    f = pl.kernel(
        _flip_compact_body,
        out_type=jax.ShapeDtypeStruct((_B, _T), jnp.float32),
        mesh=mesh,
        scratch_types=[
            pltpu.VMEM((_T,), jnp.float32),
            pltpu.VMEM((_T,), jnp.float32),
        ],
        compiler_params=pltpu.CompilerParams(needs_layout_passes=False),
    )
    x_ = f(x)
    return (x_, mask, logdet)


# speedup vs baseline: 2.9337x; 1.1204x over previous
import jax
import jax.numpy as jnp
from jax import lax
from jax.experimental import pallas as pl
from jax.experimental.pallas import tpu as pltpu
from jax.experimental.pallas import tpu_sc as plsc

_B, _T = 16, 4096
_L = 16


def _flip_compact_body(x_hbm, out_hbm, xbuf, obuf):
    c = lax.axis_index("c")
    s = lax.axis_index("s")
    wid = s * 2 + c

    @pl.when(wid < _B)
    def _():
        row = wid
        pltpu.sync_copy(x_hbm.at[row], xbuf)

        iota = lax.iota(jnp.int32, _L)
        lanes = jnp.full((_L,), _L, jnp.int32)
        tvec = jnp.full((_L,), _T, jnp.int32)

        zero = jnp.zeros((_L,), jnp.int32)

        @plsc.parallel_loop(0, _T // _L, unroll=8, carry=(zero, zero))
        def compact_step(i, carry):
            off, zoff = carry
            v = xbuf[pl.ds(_T - (i + 1) * _L, _L)]
            m = v != 0.0
            mi = m.astype(jnp.int32)
            csum = plsc.cumsum(mi)
            cnt = plsc.all_reduce_population_count(m)
            zcnt = lanes - cnt
            a = off + cnt
            b = tvec - zoff - zcnt + iota
            pos = jnp.where(m, a, b) - csum
            plsc.store_scatter(obuf, [pos], v)
            return (off + cnt, zoff + zcnt)
        pltpu.sync_copy(obuf, out_hbm.at[row])


def kernel(x, mask, logdet):
    mesh = plsc.VectorSubcoreMesh(core_axis_name="c", subcore_axis_name="s")
    f = pl.kernel(
        _flip_compact_body,
        out_type=jax.ShapeDtypeStruct((_B, _T), jnp.float32),
        mesh=mesh,
        scratch_types=[
            pltpu.VMEM((_T,), jnp.float32),
            pltpu.VMEM((_T,), jnp.float32),
        ],
        compiler_params=pltpu.CompilerParams(needs_layout_passes=False),
    )
    x_ = f(x)
    return (x_, mask, logdet)


# speedup vs baseline: 3.0709x; 1.0468x over previous
import jax
import jax.numpy as jnp
from jax import lax
from jax.experimental import pallas as pl
from jax.experimental.pallas import tpu as pltpu
from jax.experimental.pallas import tpu_sc as plsc

_B, _T = 16, 4096
_L = 16


def _flip_compact_body(x_hbm, mask_hbm, logdet_hbm, out_hbm, mask_out_hbm,
                       logdet_out_hbm, xbuf, obuf, ldbuf):
    c = lax.axis_index("c")
    s = lax.axis_index("s")
    wid = s * 2 + c

    @pl.when(wid < _B)
    def _():
        row = wid
        pltpu.sync_copy(x_hbm.at[row], xbuf)

        iota = lax.iota(jnp.int32, _L)
        lanes = jnp.full((_L,), _L, jnp.int32)
        tvec = jnp.full((_L,), _T, jnp.int32)
        zero = jnp.zeros((_L,), jnp.int32)

        @plsc.parallel_loop(0, _T // _L, unroll=8, carry=(zero, zero))
        def compact_step(i, carry):
            off, zoff = carry
            v = xbuf[pl.ds(_T - (i + 1) * _L, _L)]
            m = v != 0.0
            mi = m.astype(jnp.int32)
            csum = plsc.cumsum(mi)
            cnt = plsc.all_reduce_population_count(m)
            zcnt = lanes - cnt
            a = off + cnt
            b = tvec - zoff - zcnt + iota
            pos = jnp.where(m, a, b) - csum
            plsc.store_scatter(obuf, [pos], v)
            return (off + cnt, zoff + zcnt)

        pltpu.sync_copy(obuf, out_hbm.at[row])

    @pl.when(wid >= _B)
    def _():
        row = wid - _B
        pltpu.sync_copy(mask_hbm.at[row], xbuf)
        pltpu.sync_copy(xbuf, mask_out_hbm.at[row])

        @pl.when(row == 0)
        def _():
            pltpu.sync_copy(logdet_hbm, ldbuf)
            pltpu.sync_copy(ldbuf, logdet_out_hbm)


def kernel(x, mask, logdet):
    mesh = plsc.VectorSubcoreMesh(core_axis_name="c", subcore_axis_name="s")
    f = pl.kernel(
        _flip_compact_body,
        out_type=(
            jax.ShapeDtypeStruct((_B, _T), jnp.float32),
            jax.ShapeDtypeStruct((_B, _T), jnp.float32),
            jax.ShapeDtypeStruct((_B,), jnp.float32),
        ),
        mesh=mesh,
        scratch_types=[
            pltpu.VMEM((_T,), jnp.float32),
            pltpu.VMEM((_T,), jnp.float32),
            pltpu.VMEM((_B,), jnp.float32),
        ],
        compiler_params=pltpu.CompilerParams(needs_layout_passes=False),
    )
    x_, mask_o, logdet_o = f(x, mask, logdet)
    return (x_, mask_o, logdet_o)


# speedup vs baseline: 3.1026x; 1.0103x over previous
import jax
import jax.numpy as jnp
from jax import lax
from jax.experimental import pallas as pl
from jax.experimental.pallas import tpu as pltpu
from jax.experimental.pallas import tpu_sc as plsc

_B, _T = 16, 4096
_L = 16


def _flip_compact_body(x_hbm, mask_hbm, logdet_hbm, out_hbm, mask_out_hbm,
                       logdet_out_hbm, xbuf, obuf, ldbuf):
    c = lax.axis_index("c")
    s = lax.axis_index("s")
    wid = s * 2 + c

    @pl.when(wid < _B)
    def _():
        row = wid
        pltpu.sync_copy(x_hbm.at[row], xbuf)

        iota = lax.iota(jnp.int32, _L)
        lanes = jnp.full((_L,), _L, jnp.int32)
        tvec = jnp.full((_L,), _T, jnp.int32)
        zero = jnp.zeros((_L,), jnp.int32)

        @plsc.parallel_loop(0, _T // _L, unroll=4, carry=(zero, zero))
        def compact_step(i, carry):
            off, zoff = carry
            v = xbuf[pl.ds(_T - (i + 1) * _L, _L)]
            m = v != 0.0
            mi = m.astype(jnp.int32)
            csum = plsc.cumsum(mi)
            cnt = plsc.all_reduce_population_count(m)
            zcnt = lanes - cnt
            a = off + cnt
            b = tvec - zoff - zcnt + iota
            pos = jnp.where(m, a, b) - csum
            plsc.store_scatter(obuf, [pos], v)
            return (off + cnt, zoff + zcnt)

        pltpu.sync_copy(obuf, out_hbm.at[row])

    @pl.when(wid >= _B)
    def _():
        row = wid - _B
        pltpu.sync_copy(mask_hbm.at[row], xbuf)
        pltpu.sync_copy(xbuf, mask_out_hbm.at[row])

        @pl.when(row == 0)
        def _():
            pltpu.sync_copy(logdet_hbm, ldbuf)
            pltpu.sync_copy(ldbuf, logdet_out_hbm)


def kernel(x, mask, logdet):
    mesh = plsc.VectorSubcoreMesh(core_axis_name="c", subcore_axis_name="s")
    f = pl.kernel(
        _flip_compact_body,
        out_type=(
            jax.ShapeDtypeStruct((_B, _T), jnp.float32),
            jax.ShapeDtypeStruct((_B, _T), jnp.float32),
            jax.ShapeDtypeStruct((_B,), jnp.float32),
        ),
        mesh=mesh,
        scratch_types=[
            pltpu.VMEM((_T,), jnp.float32),
            pltpu.VMEM((_T,), jnp.float32),
            pltpu.VMEM((_B,), jnp.float32),
        ],
        compiler_params=pltpu.CompilerParams(needs_layout_passes=False),
    )
    x_, mask_o, logdet_o = f(x, mask, logdet)
    return (x_, mask_o, logdet_o)
